# baseline (speedup 1.0000x reference)
import jax
import jax.numpy as jnp
from jax.experimental import pallas as pl
from jax.experimental.pallas import tpu as pltpu


def _round_up(x, m):
    return (x + m - 1) // m * m


def _make_kernel(tm, n_pad, fo_pad):
    def _body(adj_ref, h_ref, wt_ref, o_ref, xs_ref, d_ref):
        k = pl.program_id(0)
        row0 = pl.multiple_of(k * tm, 128)

        band = adj_ref[...]
        deg = jnp.sum(band, axis=1, keepdims=True)
        d = jax.lax.rsqrt(deg + 1.0)
        d_ref[pl.ds(row0, tm), :] = d
        x = jax.lax.dot_general(
            h_ref[...], wt_ref[...],
            dimension_numbers=(((1,), (1,)), ((), ())),
            preferred_element_type=jnp.float32) * d
        xs_ref[...] = x

        part = jax.lax.dot_general(
            band, xs_ref[...],
            dimension_numbers=(((0,), (0,)), ((), ())),
            preferred_element_type=jnp.float32)

        @pl.when(k == 0)
        def _():
            o_ref[...] = part

        @pl.when(k > 0)
        def _():
            o_ref[...] += part

        o_ref[pl.ds(row0, tm), :] += xs_ref[...]

        @pl.when(k == pl.num_programs(0) - 1)
        def _():
            o_ref[...] = jnp.maximum(o_ref[...] * d_ref[...], 0.0)

    return _body


def kernel(H, adj, W):
    N, F_in = H.shape
    F_out = W.shape[0]

    n_pad = _round_up(N, 128)
    fi_pad = _round_up(F_in, 128)
    fo_pad = _round_up(F_out, 128)
    tm = 512
    while n_pad % tm:
        tm -= 128

    h_p = jnp.pad(H.astype(jnp.float32), ((0, n_pad - N), (0, fi_pad - F_in)))
    w_p = jnp.pad(W.astype(jnp.float32),
                  ((0, fo_pad - F_out), (0, fi_pad - F_in)))
    adj_p = jnp.pad(adj.astype(jnp.float32),
                    ((0, n_pad - N), (0, n_pad - N)))

    grid_rows = n_pad // tm

    out_p = pl.pallas_call(
        _make_kernel(tm, n_pad, fo_pad),
        out_shape=jax.ShapeDtypeStruct((n_pad, fo_pad), jnp.float32),
        grid_spec=pltpu.PrefetchScalarGridSpec(
            num_scalar_prefetch=0,
            grid=(grid_rows,),
            in_specs=[
                pl.BlockSpec((tm, n_pad), lambda k: (k, 0)),
                pl.BlockSpec((tm, fi_pad), lambda k: (k, 0)),
                pl.BlockSpec((fo_pad, fi_pad), lambda k: (0, 0)),
            ],
            out_specs=pl.BlockSpec((n_pad, fo_pad), lambda k: (0, 0)),
            scratch_shapes=[
                pltpu.VMEM((tm, fo_pad), jnp.float32),
                pltpu.VMEM((n_pad, 1), jnp.float32),
            ]),
        compiler_params=pltpu.CompilerParams(
            dimension_semantics=("arbitrary",),
            vmem_limit_bytes=60 * 1024 * 1024),
    )(adj_p, h_p, w_p)

    return out_p[:N, :F_out]


# speedup vs baseline: 1.0399x; 1.0399x over previous
import jax
import jax.numpy as jnp
from jax.experimental import pallas as pl
from jax.experimental.pallas import tpu as pltpu


def _round_up(x, m):
    return (x + m - 1) // m * m


def _make_kernel(tm, n_pad, fo_pad):
    def _body(adj_ref, h_ref, wt_ref, o_ref, xs_ref, d_ref):
        k = pl.program_id(0)
        row0 = pl.multiple_of(k * tm, 128)

        band = adj_ref[...]
        deg = jnp.sum(band, axis=1, keepdims=True)
        d = jax.lax.rsqrt(deg + 1.0)
        d_ref[pl.ds(row0, tm), :] = d
        x = jax.lax.dot_general(
            h_ref[...], wt_ref[...],
            dimension_numbers=(((1,), (1,)), ((), ())),
            preferred_element_type=jnp.float32) * d
        xs_ref[...] = x

        part = jax.lax.dot_general(
            band, xs_ref[...],
            dimension_numbers=(((0,), (0,)), ((), ())),
            preferred_element_type=jnp.float32)

        @pl.when(k == 0)
        def _():
            o_ref[...] = part

        @pl.when(k > 0)
        def _():
            o_ref[...] += part

        o_ref[pl.ds(row0, tm), :] += xs_ref[...]

        @pl.when(k == pl.num_programs(0) - 1)
        def _():
            o_ref[...] = jnp.maximum(o_ref[...] * d_ref[...], 0.0)

    return _body


def kernel(H, adj, W):
    N, F_in = H.shape
    F_out = W.shape[0]

    n_pad = _round_up(N, 128)
    fi_pad = _round_up(F_in, 128)
    fo_pad = _round_up(F_out, 128)
    tm = 1024
    while n_pad % tm:
        tm -= 128

    h_p = jnp.pad(H.astype(jnp.float32), ((0, n_pad - N), (0, fi_pad - F_in)))
    w_p = jnp.pad(W.astype(jnp.float32),
                  ((0, fo_pad - F_out), (0, fi_pad - F_in)))
    adj_p = jnp.pad(adj.astype(jnp.float32),
                    ((0, n_pad - N), (0, n_pad - N)))

    grid_rows = n_pad // tm

    out_p = pl.pallas_call(
        _make_kernel(tm, n_pad, fo_pad),
        out_shape=jax.ShapeDtypeStruct((n_pad, fo_pad), jnp.float32),
        grid_spec=pltpu.PrefetchScalarGridSpec(
            num_scalar_prefetch=0,
            grid=(grid_rows,),
            in_specs=[
                pl.BlockSpec((tm, n_pad), lambda k: (k, 0)),
                pl.BlockSpec((tm, fi_pad), lambda k: (k, 0)),
                pl.BlockSpec((fo_pad, fi_pad), lambda k: (0, 0)),
            ],
            out_specs=pl.BlockSpec((n_pad, fo_pad), lambda k: (0, 0)),
            scratch_shapes=[
                pltpu.VMEM((tm, fo_pad), jnp.float32),
                pltpu.VMEM((n_pad, 1), jnp.float32),
            ]),
        compiler_params=pltpu.CompilerParams(
            dimension_semantics=("arbitrary",),
            vmem_limit_bytes=60 * 1024 * 1024),
    )(adj_p, h_p, w_p)

    return out_p[:N, :F_out]
